# baseline (speedup 1.0000x reference)
import functools

import jax
import jax.numpy as jnp
from jax import lax
from jax.experimental import pallas as pl
from jax.experimental.pallas import tpu as pltpu
from jax.experimental.pallas import tpu_sc as plsc

NC = 2
NS = 16
NW = NC * NS
K = 128
CHB = 40
NBUF = 2


@functools.lru_cache(maxsize=2)
def _build(n, e, d):
    nbt = -(-e // (NW * CHB * K)) * (NW * CHB)
    nbw = nbt // NW
    e_pad = nbt * K
    n_pad = -(-(n + 1) // 2048) * 2048
    nhr = n_pad // 128
    hz = nhr // NS
    nrt = n_pad // NS

    mesh = plsc.VectorSubcoreMesh(
        core_axis_name="c", subcore_axis_name="s", num_cores=NC, num_subcores=NS
    )

    nbh = nbt // NS
    nsr = n_pad // NW

    @functools.partial(
        pl.kernel,
        out_type=(jax.ShapeDtypeStruct((NC, 2, nhr, 128), jnp.float32),
                  jax.ShapeDtypeStruct((n_pad, d), jnp.float32)),
        mesh=mesh,
        compiler_params=pltpu.CompilerParams(needs_layout_passes=False),
        scratch_types=[
            pltpu.VMEM((CHB, K), jnp.int32),
            pltpu.VMEM((CHB, K), jnp.int32),
            pltpu.VMEM((nhr, 128), jnp.float32),
            pltpu.VMEM((nhr, 128), jnp.float32),
            pltpu.VMEM((nhr, 128), jnp.float32),
            pltpu.VMEM((80, d), jnp.float32),
            pltpu.VMEM((1, nhr), jnp.int32),
            pltpu.MemorySpace.VMEM_SHARED((nhr, 128), jnp.float32),
            pltpu.MemorySpace.VMEM_SHARED((nhr, 128), jnp.float32),
        ],
    )
    def dps_kernel(heat, srcr, dstr, rowidx, out, xs_out, idx_s, idx_d,
                   hist_s, hist_d, snv, hbuf, rowidx_v, sh_s, sh_d):
        c = lax.axis_index("c")
        s = lax.axis_index("s")
        wid = c * NS + s
        pltpu.sync_copy(rowidx, rowidx_v)

        zero = jnp.zeros((16,), jnp.float32)

        def zbody(r, carry):
            def zinner(cc, carry2):
                hist_s[r, pl.ds(cc * 16, 16)] = zero
                hist_d[r, pl.ds(cc * 16, 16)] = zero
                return carry2

            return lax.fori_loop(0, 128 // 16, zinner, carry)

        lax.fori_loop(0, nhr, zbody, 0)

        @pl.when(s < nhr // 8)
        def _():
            zoff = pl.multiple_of(s * 8, 8)
            pltpu.sync_copy(hist_s.at[pl.ds(zoff, 8)], sh_s.at[pl.ds(zoff, 8)])
            pltpu.sync_copy(hist_d.at[pl.ds(zoff, 8)], sh_d.at[pl.ds(zoff, 8)])

        plsc.subcore_barrier()

        ones = jnp.ones((16,), jnp.float32)
        strip = pl.multiple_of(s * nbh, 8)
        for ch in range(nbh // CHB):
            coff = pl.multiple_of(strip + ch * CHB, 8)
            pltpu.sync_copy(srcr.at[pl.ds(coff, CHB)], idx_s)
            pltpu.sync_copy(dstr.at[pl.ds(coff, CHB)], idx_d)

            def hbody(b, carry):
                def inner(cc, carry2):
                    vs = idx_s[b, pl.ds(cc * 16, 16)]
                    plsc.addupdate_scatter(hist_s, [vs >> 7, vs & 127], ones)
                    vd = idx_d[b, pl.ds(cc * 16, 16)]
                    plsc.addupdate_scatter(hist_d, [vd >> 7, vd & 127], ones)
                    return carry2

                return lax.fori_loop(0, K // 16, inner, carry)

            lax.fori_loop(0, CHB, hbody, 0)

        pltpu.sync_copy(hist_s, sh_s.at[rowidx_v.at[0]], add=True)
        pltpu.sync_copy(hist_d, sh_d.at[rowidx_v.at[0]], add=True)
        plsc.subcore_barrier()

        @pl.when(s == 0)
        def _():
            pltpu.sync_copy(sh_s, out.at[c, 0])
            pltpu.sync_copy(sh_d, out.at[c, 1])

        pltpu.sync_copy(sh_s, snv)

        def rbody(r, carry):
            def rinner(cc, carry2):
                v = jnp.maximum(snv[r, pl.ds(cc * 16, 16)], 1.0)
                xh = 0.5 * v
                i = plsc.bitcast(v, jnp.int32)
                y = plsc.bitcast(0x5F3759DF - (i >> 1), jnp.float32)
                y = y * (1.5 - xh * y * y)
                y = y * (1.5 - xh * y * y)
                y = y * (1.5 - xh * y * y)
                snv[r, pl.ds(cc * 16, 16)] = y
                return carry2

            return lax.fori_loop(0, 128 // 16, rinner, carry)

        lax.fori_loop(0, nhr, rbody, 0)

        for t in range(nsr // 80):
            g = pl.multiple_of(wid * nsr + t * 80, 8)
            gc = wid * (nsr // 80) + t

            @pl.when(gc < n // 80)
            def _():
                pltpu.sync_copy(heat.at[pl.ds(g, 80)], hbuf)

                def sbody(ck, carry):
                    f0 = wid * nsr + t * 80 + ck * 16
                    snvec = snv[f0 // 128,
                                pl.ds(pl.multiple_of(f0 % 128, 16), 16)]
                    for j in range(16):
                        sv = jnp.full((16,), snvec[j], jnp.float32)
                        k = ck * 16 + j
                        for cc in range(d // 16):
                            hbuf[k, pl.ds(cc * 16, 16)] = (
                                hbuf[k, pl.ds(cc * 16, 16)] * sv)
                    return carry

                lax.fori_loop(0, 5, sbody, 0)
                pltpu.sync_copy(hbuf, xs_out.at[pl.ds(g, 80)])

            @pl.when(gc >= n // 80)
            def _():
                def zpad(k, carry):
                    def zpi(cc, carry2):
                        hbuf[k, pl.ds(cc * 16, 16)] = zero
                        return carry2

                    return lax.fori_loop(0, d // 16, zpi, carry)

                lax.fori_loop(0, 80, zpad, 0)
                pltpu.sync_copy(hbuf, xs_out.at[pl.ds(g, 80)])

    @functools.partial(
        pl.kernel,
        out_type=jax.ShapeDtypeStruct((NC, n_pad, d), jnp.float32),
        mesh=mesh,
        compiler_params=pltpu.CompilerParams(needs_layout_passes=False),
        scratch_types=[
            pltpu.VMEM((CHB, K), jnp.int32),
            pltpu.VMEM((CHB, K), jnp.int32),
            [pltpu.VMEM((K, d), jnp.float32) for _ in range(NBUF)],
            [pltpu.SemaphoreType.DMA for _ in range(NBUF)],
            pltpu.MemorySpace.VMEM_SHARED((n_pad, d), jnp.float32),
        ],
    )
    def agg_kernel(xs, srcr, dstr, zhbm, out, src_c, dst_c, rows, gsems,
                   acc):
        c = lax.axis_index("c")
        s = lax.axis_index("s")
        wid = c * NS + s
        strip = pl.multiple_of(wid * nbw, 8)
        base = pl.multiple_of(s * nrt, 128)
        pltpu.sync_copy(zhbm, acc.at[pl.ds(base, nrt)])
        plsc.subcore_barrier()

        for ch in range(nbw // CHB):
            coff = pl.multiple_of(strip + ch * CHB, 8)
            pltpu.sync_copy(srcr.at[pl.ds(coff, CHB)], src_c)
            pltpu.sync_copy(dstr.at[pl.ds(coff, CHB)], dst_c)
            pltpu.async_copy(xs.at[src_c.at[0]], rows[0], gsems[0])
            pltpu.async_copy(xs.at[src_c.at[1]], rows[1], gsems[1])

            def body(j2, carry):
                for p in range(NBUF):
                    t = j2 * NBUF + p
                    pltpu.make_async_copy(xs.at[src_c.at[t]], rows[p],
                                          gsems[p]).wait()
                    pltpu.sync_copy(rows[p], acc.at[dst_c.at[t]], add=True)

                    @pl.when(t + 2 < CHB)
                    def _():
                        pltpu.async_copy(xs.at[src_c.at[t + 2]], rows[p],
                                         gsems[p])

                return carry

            lax.fori_loop(0, CHB // NBUF, body, 0)

        plsc.subcore_barrier()
        pltpu.sync_copy(acc.at[pl.ds(base, nrt)], out.at[c, pl.ds(base, nrt)])

    def layer_body(emit_next, parts, dd0, sd0, w, brow, acrow, grow,
                   btrow, aarow, ghin, out1, out2):
        dn = lax.rsqrt(jnp.maximum(dd0[...], 1.0))[:n]
        agg = (parts[0, pl.ds(0, n), :] + parts[1, pl.ds(0, n), :]) * dn
        y = jnp.dot(agg, w[...], preferred_element_type=jnp.float32) + brow[...]
        ac = acrow[...]
        y = jnp.where(y >= 0, y, y * ac)
        mu = jnp.mean(y, axis=0, keepdims=True)
        var = jnp.mean((y - mu) ** 2, axis=0, keepdims=True)
        hh = (y - mu) * lax.rsqrt(var + 1e-5) * grow[...] + btrow[...]
        aa = aarow[...]
        hh = jnp.where(hh >= 0, hh, hh * aa)
        gh = jnp.sum(hh, axis=0, keepdims=True)
        if emit_next:
            sn = lax.rsqrt(jnp.maximum(sd0[...], 1.0))[:n]
            out1[pl.ds(0, n), :] = hh * sn
            out1[pl.ds(n, n_pad - n), :] = jnp.zeros((n_pad - n, d), jnp.float32)
        else:
            out1[...] = hh
        out2[...] = gh + ghin[...]

    layer1_call = pl.pallas_call(
        functools.partial(layer_body, True),
        out_shape=(jax.ShapeDtypeStruct((n_pad, d), jnp.float32),
                   jax.ShapeDtypeStruct((1, d), jnp.float32)),
        compiler_params=pltpu.CompilerParams(vmem_limit_bytes=100 * 1024 * 1024),
    )
    layer2_call = pl.pallas_call(
        functools.partial(layer_body, False),
        out_shape=(jax.ShapeDtypeStruct((n, d), jnp.float32),
                   jax.ShapeDtypeStruct((1, d), jnp.float32)),
        compiler_params=pltpu.CompilerParams(vmem_limit_bytes=100 * 1024 * 1024),
    )

    return (dps_kernel, agg_kernel, layer1_call, layer2_call,
            nbt, e_pad, n_pad, nhr)


def kernel(heat, edge_index, W1, b1, a_conv1, gamma0, beta0, a_act0,
           W2, b2, a_conv2, gamma1, beta1, a_act1):
    n, d = heat.shape
    e = edge_index.shape[1]
    (dps_kernel, agg_kernel, layer1_call, layer2_call,
     nbt, e_pad, n_pad, nhr) = _build(n, e, d)

    pad = e_pad - e
    fill = n + jnp.arange(pad, dtype=jnp.int32) % (n_pad - n)
    srcr = jnp.concatenate([edge_index[0], fill]).reshape(nbt, K)
    dstr = jnp.concatenate([edge_index[1], fill]).reshape(nbt, K)
    rowidx = jnp.arange(nhr, dtype=jnp.int32).reshape(1, nhr)
    zer = jnp.zeros((n_pad // NS, d), jnp.float32)

    degs, xs1 = dps_kernel(heat, srcr, dstr, rowidx)
    sd0 = degs[0, 0].reshape(n_pad, 1)
    dd0 = degs[0, 1].reshape(n_pad, 1)

    b1r = b1.reshape(1, d)
    g0r = gamma0.reshape(1, d)
    bt0r = beta0.reshape(1, d)
    b2r = b2.reshape(1, d)
    g1r = gamma1.reshape(1, d)
    bt1r = beta1.reshape(1, d)
    ac1r = jnp.full((1, d), a_conv1, jnp.float32)
    aa0r = jnp.full((1, d), a_act0, jnp.float32)
    ac2r = jnp.full((1, d), a_conv2, jnp.float32)
    aa1r = jnp.full((1, d), a_act1, jnp.float32)
    gh0 = jnp.zeros((1, d), jnp.float32)

    parts1 = agg_kernel(xs1, srcr, dstr, zer)
    xs2, gh1 = layer1_call(parts1, dd0, sd0, W1, b1r, ac1r, g0r,
                           bt0r, aa0r, gh0)
    parts2 = agg_kernel(xs2, srcr, dstr, zer)
    h, gh = layer2_call(parts2, dd0, sd0, W2, b2r, ac2r, g1r,
                        bt1r, aa1r, gh1)
    return (h, gh)


# speedup vs baseline: 1.0084x; 1.0084x over previous
import functools

import jax
import jax.numpy as jnp
from jax import lax
from jax.experimental import pallas as pl
from jax.experimental.pallas import tpu as pltpu
from jax.experimental.pallas import tpu_sc as plsc

NC = 2
NS = 16
NW = NC * NS
K = 128
CHB = 40
NBUF = 2


@functools.lru_cache(maxsize=2)
def _build(n, e, d):
    nbt = -(-e // (NW * CHB * K)) * (NW * CHB)
    nbw = nbt // NW
    e_pad = nbt * K
    n_pad = -(-(n + 1) // 2048) * 2048
    nhr = n_pad // 128
    hz = nhr // NS
    nrt = n_pad // NS

    mesh = plsc.VectorSubcoreMesh(
        core_axis_name="c", subcore_axis_name="s", num_cores=NC, num_subcores=NS
    )

    @functools.partial(
        pl.kernel,
        out_type=jax.ShapeDtypeStruct((NC, 2, nhr, 128), jnp.float32),
        mesh=mesh,
        compiler_params=pltpu.CompilerParams(needs_layout_passes=False),
        scratch_types=[
            pltpu.VMEM((nbw, K), jnp.int32),
            pltpu.VMEM((nbw, K), jnp.int32),
            pltpu.VMEM((nhr, 128), jnp.float32),
            pltpu.VMEM((nhr, 128), jnp.float32),
            pltpu.VMEM((1, nhr), jnp.int32),
            pltpu.MemorySpace.VMEM_SHARED((nhr, 128), jnp.float32),
            pltpu.MemorySpace.VMEM_SHARED((nhr, 128), jnp.float32),
        ],
    )
    def deg_kernel(srcr, dstr, rowidx, out, idx_s, idx_d, hist_s, hist_d,
                   rowidx_v, sh_s, sh_d):
        c = lax.axis_index("c")
        s = lax.axis_index("s")
        wid = c * NS + s
        strip = pl.multiple_of(wid * nbw, 8)
        pltpu.sync_copy(srcr.at[pl.ds(strip, nbw)], idx_s)
        pltpu.sync_copy(dstr.at[pl.ds(strip, nbw)], idx_d)
        pltpu.sync_copy(rowidx, rowidx_v)

        zero = jnp.zeros((16,), jnp.float32)

        def zbody(r, carry):
            def zinner(cc, carry2):
                hist_s[r, pl.ds(cc * 16, 16)] = zero
                hist_d[r, pl.ds(cc * 16, 16)] = zero
                return carry2

            return lax.fori_loop(0, 128 // 16, zinner, carry)

        lax.fori_loop(0, nhr, zbody, 0)

        @pl.when(s < nhr // 8)
        def _():
            zoff = pl.multiple_of(s * 8, 8)
            pltpu.sync_copy(hist_s.at[pl.ds(zoff, 8)], sh_s.at[pl.ds(zoff, 8)])
            pltpu.sync_copy(hist_d.at[pl.ds(zoff, 8)], sh_d.at[pl.ds(zoff, 8)])

        plsc.subcore_barrier()

        ones = jnp.ones((16,), jnp.float32)

        def hbody(b, carry):
            def inner(cc, carry2):
                vs = idx_s[b, pl.ds(cc * 16, 16)]
                plsc.addupdate_scatter(hist_s, [vs >> 7, vs & 127], ones)
                vd = idx_d[b, pl.ds(cc * 16, 16)]
                plsc.addupdate_scatter(hist_d, [vd >> 7, vd & 127], ones)
                return carry2

            return lax.fori_loop(0, K // 16, inner, carry)

        lax.fori_loop(0, nbw, hbody, 0)

        pltpu.sync_copy(hist_s, sh_s.at[rowidx_v.at[0]], add=True)
        pltpu.sync_copy(hist_d, sh_d.at[rowidx_v.at[0]], add=True)
        plsc.subcore_barrier()

        @pl.when(s == 0)
        def _():
            pltpu.sync_copy(sh_s, out.at[c, 0])
            pltpu.sync_copy(sh_d, out.at[c, 1])

    @functools.partial(
        pl.kernel,
        out_type=jax.ShapeDtypeStruct((NC, n_pad, d), jnp.float32),
        mesh=mesh,
        compiler_params=pltpu.CompilerParams(needs_layout_passes=False),
        scratch_types=[
            pltpu.VMEM((CHB, K), jnp.int32),
            pltpu.VMEM((CHB, K), jnp.int32),
            [pltpu.VMEM((K, d), jnp.float32) for _ in range(NBUF)],
            [pltpu.SemaphoreType.DMA for _ in range(NBUF)],
            pltpu.MemorySpace.VMEM_SHARED((n_pad, d), jnp.float32),
        ],
    )
    def agg_kernel(xs, srcr, dstr, zhbm, out, src_c, dst_c, rows, gsems,
                   acc):
        c = lax.axis_index("c")
        s = lax.axis_index("s")
        wid = c * NS + s
        strip = pl.multiple_of(wid * nbw, 8)
        base = pl.multiple_of(s * nrt, 128)
        pltpu.sync_copy(zhbm, acc.at[pl.ds(base, nrt)])
        plsc.subcore_barrier()

        for ch in range(nbw // CHB):
            coff = pl.multiple_of(strip + ch * CHB, 8)
            pltpu.sync_copy(srcr.at[pl.ds(coff, CHB)], src_c)
            pltpu.sync_copy(dstr.at[pl.ds(coff, CHB)], dst_c)
            pltpu.async_copy(xs.at[src_c.at[0]], rows[0], gsems[0])
            pltpu.async_copy(xs.at[src_c.at[1]], rows[1], gsems[1])

            def body(j2, carry):
                for p in range(NBUF):
                    t = j2 * NBUF + p
                    pltpu.make_async_copy(xs.at[src_c.at[t]], rows[p],
                                          gsems[p]).wait()
                    pltpu.sync_copy(rows[p], acc.at[dst_c.at[t]], add=True)

                    @pl.when(t + 2 < CHB)
                    def _():
                        pltpu.async_copy(xs.at[src_c.at[t + 2]], rows[p],
                                         gsems[p])

                return carry

            lax.fori_loop(0, CHB // NBUF, body, 0)

        plsc.subcore_barrier()
        pltpu.sync_copy(acc.at[pl.ds(base, nrt)], out.at[c, pl.ds(base, nrt)])

    def prep_body(heat, sd0, sd1, xs_out):
        sn = lax.rsqrt(jnp.maximum(sd0[...] + sd1[...], 1.0))[:n]
        xs_out[pl.ds(0, n), :] = heat[...] * sn
        xs_out[pl.ds(n, n_pad - n), :] = jnp.zeros((n_pad - n, d), jnp.float32)

    prep_call = pl.pallas_call(
        prep_body,
        out_shape=jax.ShapeDtypeStruct((n_pad, d), jnp.float32),
        compiler_params=pltpu.CompilerParams(vmem_limit_bytes=100 * 1024 * 1024),
    )

    def layer_body(emit_next, parts, dd0, dd1, sd0, sd1, w, brow, acrow, grow,
                   btrow, aarow, ghin, out1, out2):
        dn = lax.rsqrt(jnp.maximum(dd0[...] + dd1[...], 1.0))[:n]
        agg = (parts[0, pl.ds(0, n), :] + parts[1, pl.ds(0, n), :]) * dn
        y = jnp.dot(agg, w[...], preferred_element_type=jnp.float32) + brow[...]
        ac = acrow[...]
        y = jnp.where(y >= 0, y, y * ac)
        mu = jnp.mean(y, axis=0, keepdims=True)
        var = jnp.mean(y * y, axis=0, keepdims=True) - mu * mu
        hh = (y - mu) * lax.rsqrt(var + 1e-5) * grow[...] + btrow[...]
        aa = aarow[...]
        hh = jnp.where(hh >= 0, hh, hh * aa)
        gh = jnp.sum(hh, axis=0, keepdims=True)
        if emit_next:
            sn = lax.rsqrt(jnp.maximum(sd0[...] + sd1[...], 1.0))[:n]
            out1[pl.ds(0, n), :] = hh * sn
            out1[pl.ds(n, n_pad - n), :] = jnp.zeros((n_pad - n, d), jnp.float32)
        else:
            out1[...] = hh
        out2[...] = gh + ghin[...]

    layer1_call = pl.pallas_call(
        functools.partial(layer_body, True),
        out_shape=(jax.ShapeDtypeStruct((n_pad, d), jnp.float32),
                   jax.ShapeDtypeStruct((1, d), jnp.float32)),
        compiler_params=pltpu.CompilerParams(vmem_limit_bytes=100 * 1024 * 1024),
    )
    layer2_call = pl.pallas_call(
        functools.partial(layer_body, False),
        out_shape=(jax.ShapeDtypeStruct((n, d), jnp.float32),
                   jax.ShapeDtypeStruct((1, d), jnp.float32)),
        compiler_params=pltpu.CompilerParams(vmem_limit_bytes=100 * 1024 * 1024),
    )

    return (deg_kernel, agg_kernel, prep_call, layer1_call, layer2_call,
            nbt, e_pad, n_pad, nhr)


def kernel(heat, edge_index, W1, b1, a_conv1, gamma0, beta0, a_act0,
           W2, b2, a_conv2, gamma1, beta1, a_act1):
    n, d = heat.shape
    e = edge_index.shape[1]
    (deg_kernel, agg_kernel, prep_call, layer1_call, layer2_call,
     nbt, e_pad, n_pad, nhr) = _build(n, e, d)

    pad = e_pad - e
    fill = n + jnp.arange(pad, dtype=jnp.int32) % (n_pad - n)
    srcr = jnp.concatenate([edge_index[0], fill]).reshape(nbt, K)
    dstr = jnp.concatenate([edge_index[1], fill]).reshape(nbt, K)
    rowidx = jnp.arange(nhr, dtype=jnp.int32).reshape(1, nhr)
    zer = jnp.zeros((n_pad // NS, d), jnp.float32)

    degs = deg_kernel(srcr, dstr, rowidx)
    sd0 = degs[0, 0].reshape(n_pad, 1)
    sd1 = degs[1, 0].reshape(n_pad, 1)
    dd0 = degs[0, 1].reshape(n_pad, 1)
    dd1 = degs[1, 1].reshape(n_pad, 1)

    b1r = b1.reshape(1, d)
    g0r = gamma0.reshape(1, d)
    bt0r = beta0.reshape(1, d)
    b2r = b2.reshape(1, d)
    g1r = gamma1.reshape(1, d)
    bt1r = beta1.reshape(1, d)
    ac1r = jnp.full((1, d), a_conv1, jnp.float32)
    aa0r = jnp.full((1, d), a_act0, jnp.float32)
    ac2r = jnp.full((1, d), a_conv2, jnp.float32)
    aa1r = jnp.full((1, d), a_act1, jnp.float32)
    gh0 = jnp.zeros((1, d), jnp.float32)

    xs1 = prep_call(heat, sd0, sd1)
    parts1 = agg_kernel(xs1, srcr, dstr, zer)
    xs2, gh1 = layer1_call(parts1, dd0, dd1, sd0, sd1, W1, b1r, ac1r, g0r,
                           bt0r, aa0r, gh0)
    parts2 = agg_kernel(xs2, srcr, dstr, zer)
    h, gh = layer2_call(parts2, dd0, dd1, sd0, sd1, W2, b2r, ac2r, g1r,
                        bt1r, aa1r, gh1)
    return (h, gh)
